# baseline (speedup 1.0000x reference)
import functools

import jax
import jax.numpy as jnp
from jax import lax
from jax.experimental import pallas as pl
from jax.experimental.pallas import tpu as pltpu
from jax.experimental.pallas import tpu_sc as plsc

NC = 2
NS = 16
NW = NC * NS
LANES = 16
CHUNK = 128
ZBLK = 64


def _sc_meshes():
    return plsc.VectorSubcoreMesh(core_axis_name="c", subcore_axis_name="s")


def _sc_degree(dst2d, n, n_pad, cpt):
    npad_per_tile = n_pad // NS

    @functools.partial(
        pl.kernel,
        out_type=jax.ShapeDtypeStruct((NC * n_pad,), jnp.float32),
        mesh=_sc_meshes(),
        scratch_types=[
            pltpu.VMEM((cpt, CHUNK), jnp.int32),
            pltpu.VMEM((CHUNK,), jnp.float32),
            pltpu.VMEM((npad_per_tile,), jnp.float32),
            pltpu.VMEM_SHARED((n_pad,), jnp.float32),
        ],
    )
    def deg_kernel(dst_hbm, out_hbm, didx, ones, zbuf, acc):
        c = lax.axis_index("c")
        s = lax.axis_index("s")
        wid = c * NS + s

        one = jnp.ones((LANES,), jnp.float32)
        zero = jnp.zeros((LANES,), jnp.float32)

        @pl.loop(0, CHUNK // LANES)
        def _(r):
            ones[pl.ds(r * LANES, LANES)] = one

        @pl.loop(0, npad_per_tile // LANES)
        def _(r):
            zbuf[pl.ds(r * LANES, LANES)] = zero

        pltpu.sync_copy(dst_hbm.at[pl.ds(wid * cpt, cpt)], didx)
        rbase = s * npad_per_tile
        pltpu.sync_copy(zbuf, acc.at[pl.ds(rbase, npad_per_tile)])
        plsc.subcore_barrier()

        @pl.loop(0, cpt)
        def _(i):
            pltpu.sync_copy(ones, acc.at[didx.at[i]], add=True)

        plsc.subcore_barrier()
        pltpu.sync_copy(acc.at[pl.ds(rbase, npad_per_tile)],
                        out_hbm.at[pl.ds(c * n_pad + rbase, npad_per_tile)])

    return deg_kernel(dst2d)


def _sc_propagate(g, src2d, dst2d, n, n_pad, d, cpt):
    npad_per_tile = n_pad // NS
    nbuf = 2
    ib = 16
    nb = cpt // ib
    assert cpt % ib == 0

    @functools.partial(
        pl.kernel,
        out_type=jax.ShapeDtypeStruct((NC, n_pad, d), jnp.float32),
        mesh=_sc_meshes(),
        scratch_types=[
            pltpu.VMEM((ib, CHUNK), jnp.int32),
            pltpu.VMEM((ib, CHUNK), jnp.int32),
            [pltpu.VMEM((CHUNK, d), jnp.float32)] * nbuf,
            pltpu.VMEM((ZBLK, d), jnp.float32),
            pltpu.VMEM_SHARED((n_pad, d), jnp.float32),
            [pltpu.SemaphoreType.DMA] * nbuf,
        ],
    )
    def prop_kernel(g_hbm, src_hbm, dst_hbm, out_hbm, sidx, didx, rows, zbuf, acc,
                    gsem):
        c = lax.axis_index("c")
        s = lax.axis_index("s")
        wid = c * NS + s

        zero = jnp.zeros((LANES,), jnp.float32)

        @pl.loop(0, ZBLK)
        def _(r):
            for l in range(d // LANES):
                zbuf[r, pl.ds(l * LANES, LANES)] = zero

        @pl.loop(0, npad_per_tile // ZBLK)
        def _(i):
            pltpu.sync_copy(zbuf, acc.at[pl.ds(s * npad_per_tile + i * ZBLK, ZBLK)])

        plsc.subcore_barrier()

        for k in range(nb):
            base = wid * cpt + k * ib
            pltpu.sync_copy(src_hbm.at[pl.ds(base, ib)], sidx)
            pltpu.sync_copy(dst_hbm.at[pl.ds(base, ib)], didx)
            for b in range(nbuf):
                pltpu.async_copy(g_hbm.at[sidx.at[b]], rows[b], gsem[b])

            @pl.loop(0, ib // nbuf)
            def _(t):
                j = t * nbuf
                for b in range(nbuf):
                    pltpu.make_async_copy(g_hbm.at[sidx.at[j + b]], rows[b],
                                          gsem[b]).wait()
                    pltpu.sync_copy(rows[b], acc.at[didx.at[j + b]], add=True)

                    @pl.when(j + b + nbuf < ib)
                    def _():
                        pltpu.async_copy(g_hbm.at[sidx.at[j + b + nbuf]],
                                         rows[b], gsem[b])

        plsc.subcore_barrier()
        rbase = s * npad_per_tile
        pltpu.sync_copy(acc.at[pl.ds(rbase, npad_per_tile)],
                        out_hbm.at[c, pl.ds(rbase, npad_per_tile)])

    return prop_kernel(g, src2d, dst2d)


def _dinv_block(deg_ref):
    deg = deg_ref[:, 0:1] + deg_ref[:, 1:2] + 1.0
    return lax.rsqrt(deg)


def _tc_matmul(x, w, br):
    n, d = x.shape

    def body(x_ref, w_ref, o_ref):
        o_ref[...] = jnp.dot(x_ref[...], w_ref[...],
                             preferred_element_type=jnp.float32)

    return pl.pallas_call(
        body,
        grid=(n // br,),
        in_specs=[
            pl.BlockSpec((br, d), lambda i: (i, 0)),
            pl.BlockSpec((d, w.shape[1]), lambda i: (0, 0)),
        ],
        out_specs=pl.BlockSpec((br, w.shape[1]), lambda i: (i, 0)),
        out_shape=jax.ShapeDtypeStruct((n, w.shape[1]), jnp.float32),
    )(x, w)


def _tc_scale(degp, u, br):
    n, d = u.shape

    def body(deg_ref, u_ref, o_ref):
        dinv = _dinv_block(deg_ref)
        o_ref[...] = u_ref[...] * dinv

    return pl.pallas_call(
        body,
        grid=(n // br,),
        in_specs=[
            pl.BlockSpec((br, 2), lambda i: (i, 0)),
            pl.BlockSpec((br, d), lambda i: (i, 0)),
        ],
        out_specs=pl.BlockSpec((br, d), lambda i: (i, 0)),
        out_shape=jax.ShapeDtypeStruct((n, d), jnp.float32),
    )(degp, u)


def _tc_mid_layer(degp, rp, g, b, w, br):
    n, d = g.shape

    def body(deg_ref, rp_ref, g_ref, b_ref, w_ref, o_ref):
        dinv = _dinv_block(deg_ref)
        h = dinv * (rp_ref[0] + rp_ref[1] + g_ref[...]) + b_ref[...]
        h = jnp.maximum(h, 0.0)
        o_ref[...] = jnp.dot(h, w_ref[...],
                             preferred_element_type=jnp.float32) * dinv

    return pl.pallas_call(
        body,
        grid=(n // br,),
        in_specs=[
            pl.BlockSpec((br, 2), lambda i: (i, 0)),
            pl.BlockSpec((2, br, d), lambda i: (0, i, 0)),
            pl.BlockSpec((br, d), lambda i: (i, 0)),
            pl.BlockSpec((1, d), lambda i: (0, 0)),
            pl.BlockSpec((d, w.shape[1]), lambda i: (0, 0)),
        ],
        out_specs=pl.BlockSpec((br, w.shape[1]), lambda i: (i, 0)),
        out_shape=jax.ShapeDtypeStruct((n, w.shape[1]), jnp.float32),
    )(degp, rp, g, b, w)


def _tc_final_layer(degp, rp, g, b, w, bo, br):
    n, d = g.shape

    def body(deg_ref, rp_ref, g_ref, b_ref, w_ref, bo_ref, o_ref):
        dinv = _dinv_block(deg_ref)
        h = dinv * (rp_ref[0] + rp_ref[1] + g_ref[...]) + b_ref[...]
        h = jnp.maximum(h, 0.0)
        o_ref[...] = jnp.dot(h, w_ref[...],
                             preferred_element_type=jnp.float32) + bo_ref[...]

    return pl.pallas_call(
        body,
        grid=(n // br,),
        in_specs=[
            pl.BlockSpec((br, 2), lambda i: (i, 0)),
            pl.BlockSpec((2, br, d), lambda i: (0, i, 0)),
            pl.BlockSpec((br, d), lambda i: (i, 0)),
            pl.BlockSpec((1, d), lambda i: (0, 0)),
            pl.BlockSpec((d, w.shape[1]), lambda i: (0, 0)),
            pl.BlockSpec((1, w.shape[1]), lambda i: (0, 0)),
        ],
        out_specs=pl.BlockSpec((br, w.shape[1]), lambda i: (i, 0)),
        out_shape=jax.ShapeDtypeStruct((n, w.shape[1]), jnp.float32),
    )(degp, rp, g, b, w, bo)


def kernel(x, edge_index, W1, b1, W2, b2, Wo, bo):
    n, d = x.shape
    e = edge_index.shape[1]

    quantum = NW * CHUNK * 8
    e_pad = ((e + quantum - 1) // quantum) * quantum
    pad = e_pad - e
    n_pad = ((n + NS * ZBLK - 1) // (NS * ZBLK)) * (NS * ZBLK)
    src = edge_index[0]
    dst = edge_index[1]
    if pad:
        pad_src = jnp.arange(pad, dtype=src.dtype) % n
        pad_dst = n + (jnp.arange(pad, dtype=dst.dtype) % (n_pad - n))
        src = jnp.concatenate([src, pad_src])
        dst = jnp.concatenate([dst, pad_dst])
    cpt = e_pad // NW // CHUNK
    src2d = src.reshape(-1, CHUNK)
    dst2d = dst.reshape(-1, CHUNK)

    br = 1000 if n % 1000 == 0 else n // 10
    u1 = _tc_matmul(x, W1, br)
    degf = _sc_degree(dst2d, n, n_pad, cpt)
    degp = degf.reshape(NC, n_pad)[:, :n].T
    g1 = _tc_scale(degp, u1, br)
    r1 = _sc_propagate(g1, src2d, dst2d, n, n_pad, d, cpt)
    g2 = _tc_mid_layer(degp, r1, g1, b1.reshape(1, -1), W2, br)
    r2 = _sc_propagate(g2, src2d, dst2d, n, n_pad, W2.shape[1], cpt)
    out = _tc_final_layer(degp, r2, g2, b2.reshape(1, -1), Wo, bo.reshape(1, -1), br)
    return out


# speedup vs baseline: 1.0408x; 1.0408x over previous
import functools

import jax
import jax.numpy as jnp
from jax import lax
from jax.experimental import pallas as pl
from jax.experimental.pallas import tpu as pltpu
from jax.experimental.pallas import tpu_sc as plsc

NC = 2
NS = 16
NW = NC * NS
LANES = 16
CHUNK = 128
ZBLK = 64


def _sc_meshes():
    return plsc.VectorSubcoreMesh(core_axis_name="c", subcore_axis_name="s")


def _sc_degree(dst2d, n, n_pad, cpt):
    npad_per_tile = n_pad // NS

    @functools.partial(
        pl.kernel,
        out_type=jax.ShapeDtypeStruct((NC * n_pad,), jnp.float32),
        mesh=_sc_meshes(),
        scratch_types=[
            pltpu.VMEM((cpt, CHUNK), jnp.int32),
            pltpu.VMEM((CHUNK,), jnp.float32),
            pltpu.VMEM((npad_per_tile,), jnp.float32),
            pltpu.VMEM_SHARED((n_pad,), jnp.float32),
        ],
    )
    def deg_kernel(dst_hbm, out_hbm, didx, ones, zbuf, acc):
        c = lax.axis_index("c")
        s = lax.axis_index("s")
        wid = c * NS + s

        one = jnp.ones((LANES,), jnp.float32)
        zero = jnp.zeros((LANES,), jnp.float32)

        @pl.loop(0, CHUNK // LANES)
        def _(r):
            ones[pl.ds(r * LANES, LANES)] = one

        @pl.loop(0, npad_per_tile // LANES)
        def _(r):
            zbuf[pl.ds(r * LANES, LANES)] = zero

        pltpu.sync_copy(dst_hbm.at[pl.ds(wid * cpt, cpt)], didx)
        rbase = s * npad_per_tile
        pltpu.sync_copy(zbuf, acc.at[pl.ds(rbase, npad_per_tile)])
        plsc.subcore_barrier()

        @pl.loop(0, cpt)
        def _(i):
            pltpu.sync_copy(ones, acc.at[didx.at[i]], add=True)

        plsc.subcore_barrier()
        pltpu.sync_copy(acc.at[pl.ds(rbase, npad_per_tile)],
                        out_hbm.at[pl.ds(c * n_pad + rbase, npad_per_tile)])

    return deg_kernel(dst2d)


def _sc_propagate(g, src2d, dst2d, n, n_pad, d, cpt):
    npad_per_tile = n_pad // NS
    nbuf = 2
    ib = 16
    nb = cpt // ib
    assert cpt % ib == 0

    @functools.partial(
        pl.kernel,
        out_type=jax.ShapeDtypeStruct((NC, n_pad, d), jnp.float32),
        mesh=_sc_meshes(),
        scratch_types=[
            [pltpu.VMEM((ib, CHUNK), jnp.int32)] * 2,
            [pltpu.VMEM((ib, CHUNK), jnp.int32)] * 2,
            [pltpu.VMEM((CHUNK, d), jnp.float32)] * nbuf,
            pltpu.VMEM((ZBLK, d), jnp.float32),
            pltpu.VMEM_SHARED((n_pad, d), jnp.float32),
            [pltpu.SemaphoreType.DMA] * nbuf,
            [pltpu.SemaphoreType.DMA] * 2,
        ],
    )
    def prop_kernel(g_hbm, src_hbm, dst_hbm, out_hbm, sidx, didx, rows, zbuf, acc,
                    gsem, isem):
        c = lax.axis_index("c")
        s = lax.axis_index("s")
        wid = c * NS + s

        zero = jnp.zeros((LANES,), jnp.float32)

        @pl.loop(0, ZBLK)
        def _(r):
            for l in range(d // LANES):
                zbuf[r, pl.ds(l * LANES, LANES)] = zero

        def load_idx(k):
            base = wid * cpt + k * ib
            p = k % 2
            pltpu.async_copy(src_hbm.at[pl.ds(base, ib)], sidx[p], isem[p])
            pltpu.async_copy(dst_hbm.at[pl.ds(base, ib)], didx[p], isem[p])

        def wait_idx(k):
            base = wid * cpt + k * ib
            p = k % 2
            pltpu.make_async_copy(src_hbm.at[pl.ds(base, ib)], sidx[p],
                                  isem[p]).wait()
            pltpu.make_async_copy(dst_hbm.at[pl.ds(base, ib)], didx[p],
                                  isem[p]).wait()

        load_idx(0)

        @pl.loop(0, npad_per_tile // ZBLK)
        def _(i):
            pltpu.sync_copy(zbuf, acc.at[pl.ds(s * npad_per_tile + i * ZBLK, ZBLK)])

        wait_idx(0)
        for b in range(nbuf):
            pltpu.async_copy(g_hbm.at[sidx[0].at[b]], rows[b], gsem[b])

        plsc.subcore_barrier()

        for k in range(nb):
            p = k % 2
            if k + 1 < nb:
                load_idx(k + 1)

            @pl.loop(0, ib // nbuf)
            def _(t):
                j = t * nbuf
                for b in range(nbuf):
                    pltpu.make_async_copy(g_hbm.at[sidx[p].at[j + b]], rows[b],
                                          gsem[b]).wait()
                    pltpu.sync_copy(rows[b], acc.at[didx[p].at[j + b]], add=True)

                    @pl.when(j + b + nbuf < ib)
                    def _():
                        pltpu.async_copy(g_hbm.at[sidx[p].at[j + b + nbuf]],
                                         rows[b], gsem[b])

            if k + 1 < nb:
                wait_idx(k + 1)
                for b in range(nbuf):
                    pltpu.async_copy(g_hbm.at[sidx[(k + 1) % 2].at[b]], rows[b],
                                     gsem[b])

        plsc.subcore_barrier()
        rbase = s * npad_per_tile
        pltpu.sync_copy(acc.at[pl.ds(rbase, npad_per_tile)],
                        out_hbm.at[c, pl.ds(rbase, npad_per_tile)])

    return prop_kernel(g, src2d, dst2d)


def _dinv_block(deg_ref):
    deg = deg_ref[:, 0:1] + deg_ref[:, 1:2] + 1.0
    return lax.rsqrt(deg)


def _tc_matmul(x, w, br):
    n, d = x.shape

    def body(x_ref, w_ref, o_ref):
        o_ref[...] = jnp.dot(x_ref[...], w_ref[...],
                             preferred_element_type=jnp.float32)

    return pl.pallas_call(
        body,
        grid=(n // br,),
        in_specs=[
            pl.BlockSpec((br, d), lambda i: (i, 0)),
            pl.BlockSpec((d, w.shape[1]), lambda i: (0, 0)),
        ],
        out_specs=pl.BlockSpec((br, w.shape[1]), lambda i: (i, 0)),
        out_shape=jax.ShapeDtypeStruct((n, w.shape[1]), jnp.float32),
    )(x, w)


def _tc_scale(degp, u, br):
    n, d = u.shape

    def body(deg_ref, u_ref, o_ref):
        dinv = _dinv_block(deg_ref)
        o_ref[...] = u_ref[...] * dinv

    return pl.pallas_call(
        body,
        grid=(n // br,),
        in_specs=[
            pl.BlockSpec((br, 2), lambda i: (i, 0)),
            pl.BlockSpec((br, d), lambda i: (i, 0)),
        ],
        out_specs=pl.BlockSpec((br, d), lambda i: (i, 0)),
        out_shape=jax.ShapeDtypeStruct((n, d), jnp.float32),
    )(degp, u)


def _tc_mid_layer(degp, rp, g, b, w, br):
    n, d = g.shape

    def body(deg_ref, rp_ref, g_ref, b_ref, w_ref, o_ref):
        dinv = _dinv_block(deg_ref)
        h = dinv * (rp_ref[0] + rp_ref[1] + g_ref[...]) + b_ref[...]
        h = jnp.maximum(h, 0.0)
        o_ref[...] = jnp.dot(h, w_ref[...],
                             preferred_element_type=jnp.float32) * dinv

    return pl.pallas_call(
        body,
        grid=(n // br,),
        in_specs=[
            pl.BlockSpec((br, 2), lambda i: (i, 0)),
            pl.BlockSpec((2, br, d), lambda i: (0, i, 0)),
            pl.BlockSpec((br, d), lambda i: (i, 0)),
            pl.BlockSpec((1, d), lambda i: (0, 0)),
            pl.BlockSpec((d, w.shape[1]), lambda i: (0, 0)),
        ],
        out_specs=pl.BlockSpec((br, w.shape[1]), lambda i: (i, 0)),
        out_shape=jax.ShapeDtypeStruct((n, w.shape[1]), jnp.float32),
    )(degp, rp, g, b, w)


def _tc_final_layer(degp, rp, g, b, w, bo, br):
    n, d = g.shape

    def body(deg_ref, rp_ref, g_ref, b_ref, w_ref, bo_ref, o_ref):
        dinv = _dinv_block(deg_ref)
        h = dinv * (rp_ref[0] + rp_ref[1] + g_ref[...]) + b_ref[...]
        h = jnp.maximum(h, 0.0)
        o_ref[...] = jnp.dot(h, w_ref[...],
                             preferred_element_type=jnp.float32) + bo_ref[...]

    return pl.pallas_call(
        body,
        grid=(n // br,),
        in_specs=[
            pl.BlockSpec((br, 2), lambda i: (i, 0)),
            pl.BlockSpec((2, br, d), lambda i: (0, i, 0)),
            pl.BlockSpec((br, d), lambda i: (i, 0)),
            pl.BlockSpec((1, d), lambda i: (0, 0)),
            pl.BlockSpec((d, w.shape[1]), lambda i: (0, 0)),
            pl.BlockSpec((1, w.shape[1]), lambda i: (0, 0)),
        ],
        out_specs=pl.BlockSpec((br, w.shape[1]), lambda i: (i, 0)),
        out_shape=jax.ShapeDtypeStruct((n, w.shape[1]), jnp.float32),
    )(degp, rp, g, b, w, bo)


def kernel(x, edge_index, W1, b1, W2, b2, Wo, bo):
    n, d = x.shape
    e = edge_index.shape[1]

    quantum = NW * CHUNK * 8
    e_pad = ((e + quantum - 1) // quantum) * quantum
    pad = e_pad - e
    n_pad = ((n + NS * ZBLK - 1) // (NS * ZBLK)) * (NS * ZBLK)
    src = edge_index[0]
    dst = edge_index[1]
    if pad:
        pad_src = jnp.arange(pad, dtype=src.dtype) % n
        pad_dst = n + (jnp.arange(pad, dtype=dst.dtype) % (n_pad - n))
        src = jnp.concatenate([src, pad_src])
        dst = jnp.concatenate([dst, pad_dst])
    cpt = e_pad // NW // CHUNK
    src2d = src.reshape(-1, CHUNK)
    dst2d = dst.reshape(-1, CHUNK)

    br = 1000 if n % 1000 == 0 else n // 10
    u1 = _tc_matmul(x, W1, br)
    degf = _sc_degree(dst2d, n, n_pad, cpt)
    degp = degf.reshape(NC, n_pad)[:, :n].T
    g1 = _tc_scale(degp, u1, br)
    r1 = _sc_propagate(g1, src2d, dst2d, n, n_pad, d, cpt)
    g2 = _tc_mid_layer(degp, r1, g1, b1.reshape(1, -1), W2, br)
    r2 = _sc_propagate(g2, src2d, dst2d, n, n_pad, W2.shape[1], cpt)
    out = _tc_final_layer(degp, r2, g2, b2.reshape(1, -1), Wo, bo.reshape(1, -1), br)
    return out


# speedup vs baseline: 1.0422x; 1.0013x over previous
import functools

import jax
import jax.numpy as jnp
from jax import lax
from jax.experimental import pallas as pl
from jax.experimental.pallas import tpu as pltpu
from jax.experimental.pallas import tpu_sc as plsc

NC = 2
NS = 16
NW = NC * NS
LANES = 16
CHUNK = 128
ZBLK = 64


def _sc_meshes():
    return plsc.VectorSubcoreMesh(core_axis_name="c", subcore_axis_name="s")


def _sc_degree(dst2d, n, n_pad, cpt):
    npad_per_tile = n_pad // NS

    @functools.partial(
        pl.kernel,
        out_type=jax.ShapeDtypeStruct((NC * n_pad,), jnp.float32),
        mesh=_sc_meshes(),
        scratch_types=[
            pltpu.VMEM((cpt, CHUNK), jnp.int32),
            pltpu.VMEM((CHUNK,), jnp.float32),
            pltpu.VMEM((npad_per_tile,), jnp.float32),
            pltpu.VMEM_SHARED((n_pad,), jnp.float32),
        ],
    )
    def deg_kernel(dst_hbm, out_hbm, didx, ones, zbuf, acc):
        c = lax.axis_index("c")
        s = lax.axis_index("s")
        wid = c * NS + s

        one = jnp.ones((LANES,), jnp.float32)
        zero = jnp.zeros((LANES,), jnp.float32)

        @pl.loop(0, CHUNK // LANES)
        def _(r):
            ones[pl.ds(r * LANES, LANES)] = one

        @pl.loop(0, npad_per_tile // LANES)
        def _(r):
            zbuf[pl.ds(r * LANES, LANES)] = zero

        pltpu.sync_copy(dst_hbm.at[pl.ds(wid * cpt, cpt)], didx)
        rbase = s * npad_per_tile
        pltpu.sync_copy(zbuf, acc.at[pl.ds(rbase, npad_per_tile)])
        plsc.subcore_barrier()

        @pl.loop(0, cpt)
        def _(i):
            pltpu.sync_copy(ones, acc.at[didx.at[i]], add=True)

        plsc.subcore_barrier()
        pltpu.sync_copy(acc.at[pl.ds(rbase, npad_per_tile)],
                        out_hbm.at[pl.ds(c * n_pad + rbase, npad_per_tile)])

    return deg_kernel(dst2d)


def _sc_propagate(g, src2d, dst2d, n, n_pad, d, cpt):
    npad_per_tile = n_pad // NS
    nbuf = 2
    ib = 16
    nb = cpt // ib
    assert cpt % ib == 0

    @functools.partial(
        pl.kernel,
        out_type=jax.ShapeDtypeStruct((NC, n_pad, d), jnp.float32),
        mesh=_sc_meshes(),
        scratch_types=[
            [pltpu.VMEM((ib, CHUNK), jnp.int32)] * 2,
            [pltpu.VMEM((ib, CHUNK), jnp.int32)] * 2,
            [pltpu.VMEM((CHUNK, d), jnp.float32)] * nbuf,
            pltpu.VMEM((ZBLK, d), jnp.float32),
            pltpu.VMEM_SHARED((n_pad, d), jnp.float32),
            [pltpu.SemaphoreType.DMA] * nbuf,
            [pltpu.SemaphoreType.DMA] * 2,
        ],
    )
    def prop_kernel(g_hbm, src_hbm, dst_hbm, out_hbm, sidx, didx, rows, zbuf, acc,
                    gsem, isem):
        c = lax.axis_index("c")
        s = lax.axis_index("s")
        wid = c * NS + s

        zero = jnp.zeros((LANES,), jnp.float32)

        @pl.loop(0, ZBLK)
        def _(r):
            for l in range(d // LANES):
                zbuf[r, pl.ds(l * LANES, LANES)] = zero

        def load_idx(k):
            base = wid * cpt + k * ib
            p = k % 2
            pltpu.async_copy(src_hbm.at[pl.ds(base, ib)], sidx[p], isem[p])
            pltpu.async_copy(dst_hbm.at[pl.ds(base, ib)], didx[p], isem[p])

        def wait_idx(k):
            base = wid * cpt + k * ib
            p = k % 2
            pltpu.make_async_copy(src_hbm.at[pl.ds(base, ib)], sidx[p],
                                  isem[p]).wait()
            pltpu.make_async_copy(dst_hbm.at[pl.ds(base, ib)], didx[p],
                                  isem[p]).wait()

        load_idx(0)

        @pl.loop(0, npad_per_tile // ZBLK)
        def _(i):
            pltpu.sync_copy(zbuf, acc.at[pl.ds(s * npad_per_tile + i * ZBLK, ZBLK)])

        wait_idx(0)
        for b in range(nbuf):
            pltpu.async_copy(g_hbm.at[sidx[0].at[b]], rows[b], gsem[b])

        plsc.subcore_barrier()

        for k in range(nb):
            p = k % 2
            if k + 1 < nb:
                load_idx(k + 1)

            @pl.loop(0, ib // nbuf)
            def _(t):
                j = t * nbuf
                for b in range(nbuf):
                    pltpu.make_async_copy(g_hbm.at[sidx[p].at[j + b]], rows[b],
                                          gsem[b]).wait()
                    pltpu.sync_copy(rows[b], acc.at[didx[p].at[j + b]], add=True)

                    @pl.when(j + b + nbuf < ib)
                    def _():
                        pltpu.async_copy(g_hbm.at[sidx[p].at[j + b + nbuf]],
                                         rows[b], gsem[b])

            if k + 1 < nb:
                wait_idx(k + 1)
                for b in range(nbuf):
                    pltpu.async_copy(g_hbm.at[sidx[(k + 1) % 2].at[b]], rows[b],
                                     gsem[b])

        plsc.subcore_barrier()
        rbase = s * npad_per_tile
        pltpu.sync_copy(acc.at[pl.ds(rbase, npad_per_tile)],
                        out_hbm.at[c, pl.ds(rbase, npad_per_tile)])

    return prop_kernel(g, src2d, dst2d)


def _dinv_block(deg_ref):
    deg = deg_ref[:, 0:1] + deg_ref[:, 1:2] + 1.0
    return lax.rsqrt(deg)


def _tc_matmul(x, w, br):
    n, d = x.shape

    def body(x_ref, w_ref, o_ref):
        o_ref[...] = jnp.dot(x_ref[...], w_ref[...],
                             preferred_element_type=jnp.float32)

    return pl.pallas_call(
        body,
        grid=(n // br,),
        in_specs=[
            pl.BlockSpec((br, d), lambda i: (i, 0)),
            pl.BlockSpec((d, w.shape[1]), lambda i: (0, 0)),
        ],
        out_specs=pl.BlockSpec((br, w.shape[1]), lambda i: (i, 0)),
        out_shape=jax.ShapeDtypeStruct((n, w.shape[1]), jnp.float32),
    )(x, w)


def _tc_scale_matmul(degp, x, w, br):
    n, d = x.shape

    def body(deg_ref, x_ref, w_ref, o_ref):
        dinv = _dinv_block(deg_ref)
        o_ref[...] = jnp.dot(x_ref[...], w_ref[...],
                             preferred_element_type=jnp.float32) * dinv

    return pl.pallas_call(
        body,
        grid=(n // br,),
        in_specs=[
            pl.BlockSpec((br, 2), lambda i: (i, 0)),
            pl.BlockSpec((br, d), lambda i: (i, 0)),
            pl.BlockSpec((d, w.shape[1]), lambda i: (0, 0)),
        ],
        out_specs=pl.BlockSpec((br, w.shape[1]), lambda i: (i, 0)),
        out_shape=jax.ShapeDtypeStruct((n, w.shape[1]), jnp.float32),
    )(degp, x, w)


def _tc_mid_layer(degp, rp, g, b, w, br):
    n, d = g.shape

    def body(deg_ref, rp_ref, g_ref, b_ref, w_ref, o_ref):
        dinv = _dinv_block(deg_ref)
        h = dinv * (rp_ref[0] + rp_ref[1] + g_ref[...]) + b_ref[...]
        h = jnp.maximum(h, 0.0)
        o_ref[...] = jnp.dot(h, w_ref[...],
                             preferred_element_type=jnp.float32) * dinv

    return pl.pallas_call(
        body,
        grid=(n // br,),
        in_specs=[
            pl.BlockSpec((br, 2), lambda i: (i, 0)),
            pl.BlockSpec((2, br, d), lambda i: (0, i, 0)),
            pl.BlockSpec((br, d), lambda i: (i, 0)),
            pl.BlockSpec((1, d), lambda i: (0, 0)),
            pl.BlockSpec((d, w.shape[1]), lambda i: (0, 0)),
        ],
        out_specs=pl.BlockSpec((br, w.shape[1]), lambda i: (i, 0)),
        out_shape=jax.ShapeDtypeStruct((n, w.shape[1]), jnp.float32),
    )(degp, rp, g, b, w)


def _tc_final_layer(degp, rp, g, b, w, bo, br):
    n, d = g.shape

    def body(deg_ref, rp_ref, g_ref, b_ref, w_ref, bo_ref, o_ref):
        dinv = _dinv_block(deg_ref)
        h = dinv * (rp_ref[0] + rp_ref[1] + g_ref[...]) + b_ref[...]
        h = jnp.maximum(h, 0.0)
        o_ref[...] = jnp.dot(h, w_ref[...],
                             preferred_element_type=jnp.float32) + bo_ref[...]

    return pl.pallas_call(
        body,
        grid=(n // br,),
        in_specs=[
            pl.BlockSpec((br, 2), lambda i: (i, 0)),
            pl.BlockSpec((2, br, d), lambda i: (0, i, 0)),
            pl.BlockSpec((br, d), lambda i: (i, 0)),
            pl.BlockSpec((1, d), lambda i: (0, 0)),
            pl.BlockSpec((d, w.shape[1]), lambda i: (0, 0)),
            pl.BlockSpec((1, w.shape[1]), lambda i: (0, 0)),
        ],
        out_specs=pl.BlockSpec((br, w.shape[1]), lambda i: (i, 0)),
        out_shape=jax.ShapeDtypeStruct((n, w.shape[1]), jnp.float32),
    )(degp, rp, g, b, w, bo)


def kernel(x, edge_index, W1, b1, W2, b2, Wo, bo):
    n, d = x.shape
    e = edge_index.shape[1]

    quantum = NW * CHUNK * 8
    e_pad = ((e + quantum - 1) // quantum) * quantum
    pad = e_pad - e
    n_pad = ((n + NS * ZBLK - 1) // (NS * ZBLK)) * (NS * ZBLK)
    src = edge_index[0]
    dst = edge_index[1]
    if pad:
        pad_src = jnp.arange(pad, dtype=src.dtype) % n
        pad_dst = n + (jnp.arange(pad, dtype=dst.dtype) % (n_pad - n))
        src = jnp.concatenate([src, pad_src])
        dst = jnp.concatenate([dst, pad_dst])
    cpt = e_pad // NW // CHUNK
    src2d = src.reshape(-1, CHUNK)
    dst2d = dst.reshape(-1, CHUNK)

    br = 1000 if n % 1000 == 0 else n // 10
    degf = _sc_degree(dst2d, n, n_pad, cpt)
    degp = degf.reshape(NC, n_pad)[:, :n].T
    g1 = _tc_scale_matmul(degp, x, W1, br)
    r1 = _sc_propagate(g1, src2d, dst2d, n, n_pad, d, cpt)
    g2 = _tc_mid_layer(degp, r1, g1, b1.reshape(1, -1), W2, br)
    r2 = _sc_propagate(g2, src2d, dst2d, n, n_pad, W2.shape[1], cpt)
    out = _tc_final_layer(degp, r2, g2, b2.reshape(1, -1), Wo, bo.reshape(1, -1), br)
    return out


# speedup vs baseline: 1.0422x; 1.0000x over previous
import functools

import jax
import jax.numpy as jnp
from jax import lax
from jax.experimental import pallas as pl
from jax.experimental.pallas import tpu as pltpu
from jax.experimental.pallas import tpu_sc as plsc

NC = 2
NS = 16
NW = NC * NS
LANES = 16
CHUNK = 128
ZBLK = 64


def _sc_meshes():
    return plsc.VectorSubcoreMesh(core_axis_name="c", subcore_axis_name="s")


def _sc_degree(dst2d, n, n_pad, cpt):
    npad_per_tile = n_pad // NS

    @functools.partial(
        pl.kernel,
        out_type=jax.ShapeDtypeStruct((NC * n_pad,), jnp.float32),
        mesh=_sc_meshes(),
        scratch_types=[
            pltpu.VMEM((cpt, CHUNK), jnp.int32),
            pltpu.VMEM((CHUNK,), jnp.float32),
            pltpu.VMEM((npad_per_tile,), jnp.float32),
            pltpu.VMEM_SHARED((n_pad,), jnp.float32),
        ],
    )
    def deg_kernel(dst_hbm, out_hbm, didx, ones, zbuf, acc):
        c = lax.axis_index("c")
        s = lax.axis_index("s")
        wid = c * NS + s

        one = jnp.ones((LANES,), jnp.float32)
        zero = jnp.zeros((LANES,), jnp.float32)

        @pl.loop(0, CHUNK // LANES)
        def _(r):
            ones[pl.ds(r * LANES, LANES)] = one

        @pl.loop(0, npad_per_tile // LANES)
        def _(r):
            zbuf[pl.ds(r * LANES, LANES)] = zero

        pltpu.sync_copy(dst_hbm.at[pl.ds(wid * cpt, cpt)], didx)
        rbase = s * npad_per_tile
        pltpu.sync_copy(zbuf, acc.at[pl.ds(rbase, npad_per_tile)])
        plsc.subcore_barrier()

        @pl.loop(0, cpt)
        def _(i):
            pltpu.sync_copy(ones, acc.at[didx.at[i]], add=True)

        plsc.subcore_barrier()
        pltpu.sync_copy(acc.at[pl.ds(rbase, npad_per_tile)],
                        out_hbm.at[pl.ds(c * n_pad + rbase, npad_per_tile)])

    return deg_kernel(dst2d)


def _sc_propagate(g, src2d, dst2d, n, n_pad, d, cpt):
    npad_per_tile = n_pad // NS
    nbuf = 2
    ib = 16
    nb = cpt // ib
    assert cpt % ib == 0

    @functools.partial(
        pl.kernel,
        out_type=jax.ShapeDtypeStruct((NC, n_pad, d), jnp.float32),
        mesh=_sc_meshes(),
        scratch_types=[
            [pltpu.VMEM((ib, CHUNK), jnp.int32)] * 2,
            [pltpu.VMEM((ib, CHUNK), jnp.int32)] * 2,
            [pltpu.VMEM((CHUNK, d), jnp.float32)] * nbuf,
            pltpu.VMEM((ZBLK, d), jnp.float32),
            pltpu.VMEM_SHARED((n_pad, d), jnp.float32),
            [pltpu.SemaphoreType.DMA] * nbuf,
            [pltpu.SemaphoreType.DMA] * 2,
        ],
    )
    def prop_kernel(g_hbm, src_hbm, dst_hbm, out_hbm, sidx, didx, rows, zbuf, acc,
                    gsem, isem):
        c = lax.axis_index("c")
        s = lax.axis_index("s")
        wid = c * NS + s

        zero = jnp.zeros((LANES,), jnp.float32)

        @pl.loop(0, ZBLK)
        def _(r):
            for l in range(d // LANES):
                zbuf[r, pl.ds(l * LANES, LANES)] = zero

        def load_idx(k):
            base = wid * cpt + k * ib
            p = k % 2
            pltpu.async_copy(src_hbm.at[pl.ds(base, ib)], sidx[p], isem[p])
            pltpu.async_copy(dst_hbm.at[pl.ds(base, ib)], didx[p], isem[p])

        def wait_idx(k):
            base = wid * cpt + k * ib
            p = k % 2
            pltpu.make_async_copy(src_hbm.at[pl.ds(base, ib)], sidx[p],
                                  isem[p]).wait()
            pltpu.make_async_copy(dst_hbm.at[pl.ds(base, ib)], didx[p],
                                  isem[p]).wait()

        load_idx(0)

        @pl.loop(0, npad_per_tile // ZBLK)
        def _(i):
            pltpu.sync_copy(zbuf, acc.at[pl.ds(s * npad_per_tile + i * ZBLK, ZBLK)])

        wait_idx(0)
        for b in range(nbuf):
            pltpu.async_copy(g_hbm.at[sidx[0].at[b]], rows[b], gsem[b])

        plsc.subcore_barrier()

        for k in range(nb):
            p = k % 2
            if k + 1 < nb:
                load_idx(k + 1)

            @pl.loop(0, ib // nbuf)
            def _(t):
                j = t * nbuf
                for b in range(nbuf):
                    pltpu.make_async_copy(g_hbm.at[sidx[p].at[j + b]], rows[b],
                                          gsem[b]).wait()
                    pltpu.sync_copy(rows[b], acc.at[didx[p].at[j + b]], add=True)

                    @pl.when(j + b + nbuf < ib)
                    def _():
                        pltpu.async_copy(g_hbm.at[sidx[p].at[j + b + nbuf]],
                                         rows[b], gsem[b])

            if k + 1 < nb:
                wait_idx(k + 1)
                for b in range(nbuf):
                    pltpu.async_copy(g_hbm.at[sidx[(k + 1) % 2].at[b]], rows[b],
                                     gsem[b])

        plsc.subcore_barrier()
        rbase = s * npad_per_tile
        pltpu.sync_copy(acc.at[pl.ds(rbase, npad_per_tile)],
                        out_hbm.at[c, pl.ds(rbase, npad_per_tile)])

    return prop_kernel(g, src2d, dst2d)


def _dinv_block(deg_ref):
    deg = deg_ref[:, 0:1] + deg_ref[:, 1:2] + 1.0
    return lax.rsqrt(deg)


def _tc_scale_matmul(degp, x, w, br):
    n, d = x.shape

    def body(deg_ref, x_ref, w_ref, o_ref):
        dinv = _dinv_block(deg_ref)
        o_ref[...] = jnp.dot(x_ref[...], w_ref[...],
                             preferred_element_type=jnp.float32) * dinv

    return pl.pallas_call(
        body,
        grid=(n // br,),
        in_specs=[
            pl.BlockSpec((br, 2), lambda i: (i, 0)),
            pl.BlockSpec((br, d), lambda i: (i, 0)),
            pl.BlockSpec((d, w.shape[1]), lambda i: (0, 0)),
        ],
        out_specs=pl.BlockSpec((br, w.shape[1]), lambda i: (i, 0)),
        out_shape=jax.ShapeDtypeStruct((n, w.shape[1]), jnp.float32),
    )(degp, x, w)


def _tc_mid_layer(degp, rp, g, b, w, br):
    n, d = g.shape

    def body(deg_ref, rp_ref, g_ref, b_ref, w_ref, o_ref):
        dinv = _dinv_block(deg_ref)
        h = dinv * (rp_ref[0] + rp_ref[1] + g_ref[...]) + b_ref[...]
        h = jnp.maximum(h, 0.0)
        o_ref[...] = jnp.dot(h, w_ref[...],
                             preferred_element_type=jnp.float32) * dinv

    return pl.pallas_call(
        body,
        grid=(n // br,),
        in_specs=[
            pl.BlockSpec((br, 2), lambda i: (i, 0)),
            pl.BlockSpec((2, br, d), lambda i: (0, i, 0)),
            pl.BlockSpec((br, d), lambda i: (i, 0)),
            pl.BlockSpec((1, d), lambda i: (0, 0)),
            pl.BlockSpec((d, w.shape[1]), lambda i: (0, 0)),
        ],
        out_specs=pl.BlockSpec((br, w.shape[1]), lambda i: (i, 0)),
        out_shape=jax.ShapeDtypeStruct((n, w.shape[1]), jnp.float32),
    )(degp, rp, g, b, w)


def _tc_final_layer(degp, rp, g, b, w, bo, br):
    n, d = g.shape

    def body(deg_ref, rp_ref, g_ref, b_ref, w_ref, bo_ref, o_ref):
        dinv = _dinv_block(deg_ref)
        h = dinv * (rp_ref[0] + rp_ref[1] + g_ref[...]) + b_ref[...]
        h = jnp.maximum(h, 0.0)
        o_ref[...] = jnp.dot(h, w_ref[...],
                             preferred_element_type=jnp.float32) + bo_ref[...]

    return pl.pallas_call(
        body,
        grid=(n // br,),
        in_specs=[
            pl.BlockSpec((br, 2), lambda i: (i, 0)),
            pl.BlockSpec((2, br, d), lambda i: (0, i, 0)),
            pl.BlockSpec((br, d), lambda i: (i, 0)),
            pl.BlockSpec((1, d), lambda i: (0, 0)),
            pl.BlockSpec((d, w.shape[1]), lambda i: (0, 0)),
            pl.BlockSpec((1, w.shape[1]), lambda i: (0, 0)),
        ],
        out_specs=pl.BlockSpec((br, w.shape[1]), lambda i: (i, 0)),
        out_shape=jax.ShapeDtypeStruct((n, w.shape[1]), jnp.float32),
    )(degp, rp, g, b, w, bo)


def kernel(x, edge_index, W1, b1, W2, b2, Wo, bo):
    n, d = x.shape
    e = edge_index.shape[1]

    quantum = NW * CHUNK * 8
    e_pad = ((e + quantum - 1) // quantum) * quantum
    pad = e_pad - e
    n_pad = ((n + NS * ZBLK - 1) // (NS * ZBLK)) * (NS * ZBLK)
    src = edge_index[0]
    dst = edge_index[1]
    if pad:
        pad_src = jnp.arange(pad, dtype=src.dtype) % n
        pad_dst = n + (jnp.arange(pad, dtype=dst.dtype) % (n_pad - n))
        src = jnp.concatenate([src, pad_src])
        dst = jnp.concatenate([dst, pad_dst])
    cpt = e_pad // NW // CHUNK
    src2d = src.reshape(-1, CHUNK)
    dst2d = dst.reshape(-1, CHUNK)

    br = 1000 if n % 1000 == 0 else n // 10
    degf = _sc_degree(dst2d, n, n_pad, cpt)
    degp = degf.reshape(NC, n_pad)[:, :n].T
    g1 = _tc_scale_matmul(degp, x, W1, br)
    r1 = _sc_propagate(g1, src2d, dst2d, n, n_pad, d, cpt)
    g2 = _tc_mid_layer(degp, r1, g1, b1.reshape(1, -1), W2, br)
    r2 = _sc_propagate(g2, src2d, dst2d, n, n_pad, W2.shape[1], cpt)
    out = _tc_final_layer(degp, r2, g2, b2.reshape(1, -1), Wo, bo.reshape(1, -1), br)
    return out
